# baseline (speedup 1.0000x reference)
import functools

import jax
import jax.numpy as jnp
from jax.experimental import pallas as pl

N = 10000
D = 128
H = 8
C = 16
DFF = 256
L = 2

ROWS = 1000


def _ln(x, g, b):
    mu = jnp.mean(x, axis=-1, keepdims=True)
    var = jnp.mean((x - mu) ** 2, axis=-1, keepdims=True)
    return (x - mu) / jnp.sqrt(var + 1e-5) * g + b


def _ffn_ln_body(x_ref, x2_ref, g1_ref, be1_ref, W1_ref, b1_ref, W2_ref, b2_ref,
                 g2_ref, be2_ref, o_ref):
    x = x_ref[...]
    x2 = x2_ref[...]
    xa = _ln(x + x2, g1_ref[...], be1_ref[...])
    h = jnp.maximum(
        jnp.dot(xa, W1_ref[...], preferred_element_type=jnp.float32,
                precision=jax.lax.Precision.HIGHEST) + b1_ref[...], 0.0)
    h = jnp.dot(h, W2_ref[...], preferred_element_type=jnp.float32,
                precision=jax.lax.Precision.HIGHEST) + b2_ref[...]
    o_ref[...] = _ln(xa + h, g2_ref[...], be2_ref[...])


@functools.partial(jax.jit, static_argnums=())
def _ffn_ln(x, x2, g1, be1, W1, b1, W2, b2, g2, be2):
    grid = (N // ROWS,)
    row_spec = pl.BlockSpec((ROWS, D), lambda i: (i, 0))
    full = lambda s: pl.BlockSpec(s, lambda i: (0,) * len(s))
    return pl.pallas_call(
        _ffn_ln_body,
        grid=grid,
        in_specs=[row_spec, row_spec, full((D,)), full((D,)), full((D, DFF)),
                  full((DFF,)), full((DFF, D)), full((D,)), full((D,)), full((D,))],
        out_specs=row_spec,
        out_shape=jax.ShapeDtypeStruct((N, D), jnp.float32),
    )(x, x2, g1, be1, W1, b1, W2, b2, g2, be2)


def _segment_softmax(alpha, seg, n):
    amax = jax.ops.segment_max(alpha, seg, num_segments=n)
    amax = jnp.where(jnp.isfinite(amax), amax, 0.0)
    ex = jnp.exp(alpha - amax[seg])
    den = jax.ops.segment_sum(ex, seg, num_segments=n)
    return ex / (den[seg] + 1e-16)


def kernel(input, embedding, edge_attr, edge_index, W_in, b_in, W_emb, b_emb,
           Wq, bq, Wk, bk, Wv, bv, We, be, Ws, bs, W1, b1, W2, b2, g1, be1,
           g2, be2):
    src = edge_index[0]
    dst = edge_index[1]
    inp = input @ W_in + b_in
    x = embedding @ W_emb + b_emb
    for l in range(L):
        x = x + inp
        q = (x @ Wq[l] + bq[l]).reshape(N, H, C)
        k = (x @ Wk[l] + bk[l]).reshape(N, H, C)
        v = (x @ Wv[l] + bv[l]).reshape(N, H, C)
        eemb = (edge_attr @ We[l] + be[l]).reshape(-1, H, C)
        kj = k[src] + eemb
        vj = v[src] + eemb
        alpha = jnp.sum(q[dst] * kj, axis=-1) / jnp.sqrt(float(C))
        alpha = _segment_softmax(alpha, dst, N)
        out = jax.ops.segment_sum(vj * alpha[:, :, None], dst,
                                  num_segments=N).reshape(N, H * C)
        x2 = out + x @ Ws[l] + bs[l]
        x = _ffn_ln(x, x2, g1[l], be1[l], W1[l], b1[l], W2[l], b2[l],
                    g2[l], be2[l])
    return x


# speedup vs baseline: 12.0201x; 12.0201x over previous
import functools

import jax
import jax.numpy as jnp
from jax import lax
from jax.experimental import pallas as pl
from jax.experimental.pallas import tpu as pltpu
from jax.experimental.pallas import tpu_sc as plsc

N = 10000
E = 320000
D = 128
H = 8
C = 16
DFF = 256
DE = 16
L = 2

ROWS = 1000
HP = jax.lax.Precision.HIGHEST

NC = 2
NS = 16
LANES = 16
HC = H // NC
CH = 40
EPS = E // NS
NCH = EPS // CH
RSTEP = 624
RWIN = 640
AW = 128
DNR = 640


def _ln(x, g, b):
    mu = jnp.mean(x, axis=-1, keepdims=True)
    var = jnp.mean((x - mu) ** 2, axis=-1, keepdims=True)
    return (x - mu) / jnp.sqrt(var + 1e-5) * g + b


def _proj_body(a_ref, b_ref, Wa_ref, ba_ref, Wb_ref, bb_ref, inp_ref, x0_ref):
    inp_ref[...] = jnp.dot(a_ref[...], Wa_ref[...],
                           preferred_element_type=jnp.float32,
                           precision=HP) + ba_ref[...]
    x0_ref[...] = jnp.dot(b_ref[...], Wb_ref[...],
                          preferred_element_type=jnp.float32,
                          precision=HP) + bb_ref[...]


def _proj(input, embedding, W_in, b_in, W_emb, b_emb):
    row = pl.BlockSpec((ROWS, D), lambda i: (i, 0))
    full = lambda s: pl.BlockSpec(s, lambda i: (0,) * len(s))
    return pl.pallas_call(
        _proj_body,
        grid=(N // ROWS,),
        in_specs=[row, row, full((D, D)), full((D,)), full((D, D)), full((D,))],
        out_specs=[row, row],
        out_shape=[jax.ShapeDtypeStruct((N, D), jnp.float32),
                   jax.ShapeDtypeStruct((N, D), jnp.float32)],
    )(input, embedding, W_in, b_in, W_emb, b_emb)


def _tables_body(x_ref, inp_ref, Wcat_ref, bcat_ref, y_ref, qq_ref, kv_ref):
    y = x_ref[...] + inp_ref[...]
    y_ref[...] = y
    T = jnp.dot(y, Wcat_ref[...], preferred_element_type=jnp.float32,
                precision=HP) + bcat_ref[...]
    qq_ref[0] = T[:, 0:128]
    qq_ref[1] = T[:, 128:256]
    kv_ref[0] = T[:, 256:384]
    kv_ref[1] = T[:, 384:512]


def _tables(x, inp, Wcat, bcat):
    row = pl.BlockSpec((ROWS, D), lambda i: (i, 0))
    out2 = pl.BlockSpec((2, ROWS, D), lambda i: (0, i, 0))
    full = lambda s: pl.BlockSpec(s, lambda i: (0,) * len(s))
    return pl.pallas_call(
        _tables_body,
        grid=(N // ROWS,),
        in_specs=[row, row, full((D, 4 * D)), full((4 * D,))],
        out_specs=[row, out2, out2],
        out_shape=[jax.ShapeDtypeStruct((N, D), jnp.float32),
                   jax.ShapeDtypeStruct((2, N, D), jnp.float32),
                   jax.ShapeDtypeStruct((2, N, D), jnp.float32)],
    )(x, inp, Wcat, bcat)


def _edge_body(qq_hbm, kv_hbm, src_hbm, dst_hbm, ea_hbm, out_hbm, den_hbm,
               dstb, srcb, qqib, kvib, dnib, eab, qqr, kvr, sb, sb2,
               acc, dacc, sem1, sem2):
    c = lax.axis_index("c")
    s = lax.axis_index("s")
    cN = (c * N).astype(jnp.int32)
    zv = jnp.zeros((LANES,), jnp.float32)
    lane = lax.iota(jnp.int32, LANES)

    def zrow(i, _):
        for j in range(AW // LANES):
            sb[i, pl.ds(j * LANES, LANES)] = zv
            sb2[i, pl.ds(j * LANES, LANES)] = zv
        return 0
    lax.fori_loop(0, CH, zrow, 0)
    rbase = s * RSTEP
    for t in range(RWIN // CH):
        pltpu.sync_copy(sb, acc.at[pl.ds(rbase + t * CH, CH)])
    pltpu.sync_copy(sb.at[pl.ds(0, DNR // NS)],
                    dacc.at[pl.ds(s * (DNR // NS), DNR // NS)])
    plsc.subcore_barrier()

    ebase = s * EPS

    def chunk(j, _):
        off = ebase + j * CH
        pltpu.sync_copy(dst_hbm.at[pl.ds(off, CH)], dstb)
        pltpu.sync_copy(src_hbm.at[pl.ds(off, CH)], srcb)
        pltpu.sync_copy(ea_hbm.at[pl.ds(off, CH)], eab)
        for t in range(CH // LANES + 1):
            sl = pl.ds(min(t * LANES, CH - LANES), LANES)
            qqib[sl] = dstb[sl] + cN
            kvib[sl] = srcb[sl] + cN
            dnib[sl] = lax.shift_right_logical(dstb[sl], 4)
        gq = pltpu.async_copy(qq_hbm.at[qqib], qqr, sem1)
        gk = pltpu.async_copy(kv_hbm.at[kvib], kvr, sem2)
        gq.wait()
        gk.wait()
        dv0 = dstb[pl.ds(0, LANES)]
        dv1 = dstb[pl.ds(16, LANES)]
        dv2 = dstb[pl.ds(CH - LANES, LANES)]

        def edge(e, dvs):
            d0, d1, d2 = dvs
            ea_v = eab[e, :]
            dv = jnp.where(e < 16, d0, jnp.where(e < 32, d1, d2))
            pos = jnp.where(e < 32, e & 15, e - (CH - LANES))
            de_vec = jnp.take_along_axis(
                dv, jnp.broadcast_to(pos, (LANES,)), axis=0)
            par8 = (de_vec & 1) * 8
            den = zv
            for h in range(HC):
                qv = qqr[e, pl.ds(h * LANES, LANES)]
                qev = qqr[e, pl.ds(64 + h * LANES, LANES)]
                kvv = kvr[e, pl.ds(h * LANES, LANES)]
                vv = kvr[e, pl.ds(64 + h * LANES, LANES)]
                t_ = qv * kvv + qev * ea_v
                a = jnp.sum(t_)
                s_vec = jnp.exp(jnp.broadcast_to(a, (LANES,)))
                sb[e, pl.ds(h * LANES, LANES)] = s_vec * vv
                sb[e, pl.ds(64 + h * LANES, LANES)] = s_vec * ea_v
                den = jnp.where(lane == h + par8, s_vec, den)
            for t in range(AW // LANES):
                sb2[e, pl.ds(t * LANES, LANES)] = zv
            col = (lax.shift_right_logical(de_vec, 1) & 7) * LANES + lane
            row = jnp.broadcast_to(e, (LANES,))
            plsc.store_scatter(sb2, [row, col], den)
            return dvs
        lax.fori_loop(0, CH, edge, (dv0, dv1, dv2))
        pltpu.sync_copy(sb, acc.at[dstb], add=True)
        pltpu.sync_copy(sb2, dacc.at[dnib], add=True)
        return 0
    lax.fori_loop(0, NCH, chunk, 0)

    plsc.subcore_barrier()
    pltpu.sync_copy(acc.at[pl.ds(rbase, RWIN)],
                    out_hbm.at[c, pl.ds(rbase, RWIN)])
    pltpu.sync_copy(dacc.at[pl.ds(s * (DNR // NS), DNR // NS)],
                    den_hbm.at[c, pl.ds(s * (DNR // NS), DNR // NS)])


def _edge_phase(qq, kv, src, dst, edge_attr):
    mesh = plsc.VectorSubcoreMesh(core_axis_name="c", subcore_axis_name="s")
    f = functools.partial(
        pl.kernel,
        mesh=mesh,
        compiler_params=pltpu.CompilerParams(needs_layout_passes=False),
        out_type=[jax.ShapeDtypeStruct((2, N, AW), jnp.float32),
                  jax.ShapeDtypeStruct((2, DNR, AW), jnp.float32)],
        scratch_types=[
            pltpu.VMEM((CH,), jnp.int32),
            pltpu.VMEM((CH,), jnp.int32),
            pltpu.VMEM((CH,), jnp.int32),
            pltpu.VMEM((CH,), jnp.int32),
            pltpu.VMEM((CH,), jnp.int32),
            pltpu.VMEM((CH, DE), jnp.float32),
            pltpu.VMEM((CH, D), jnp.float32),
            pltpu.VMEM((CH, D), jnp.float32),
            pltpu.VMEM((CH, AW), jnp.float32),
            pltpu.VMEM((CH, AW), jnp.float32),
            pltpu.VMEM_SHARED((N, AW), jnp.float32),
            pltpu.VMEM_SHARED((DNR, AW), jnp.float32),
            pltpu.SemaphoreType.DMA,
            pltpu.SemaphoreType.DMA,
        ],
    )(_edge_body)
    return f(qq, kv, src, dst, edge_attr)


def _post_body(acc_ref, den_ref, y_ref, Wz_ref, S_ref, bev_ref, Ws_ref, bs_ref,
               W1_ref, b1_ref, W2_ref, b2_ref, g1_ref, be1_ref, g2_ref,
               be2_ref, o_ref):
    acc0 = acc_ref[0]
    acc1 = acc_ref[1]
    outv = jnp.concatenate([acc0[:, 0:64], acc1[:, 0:64]], axis=1)
    z = jnp.concatenate([acc0[:, 64:128], acc1[:, 64:128]], axis=1)
    den_rep = jnp.dot(den_ref[...], S_ref[...],
                      preferred_element_type=jnp.float32, precision=HP)
    num = outv + jnp.dot(z, Wz_ref[...], preferred_element_type=jnp.float32,
                         precision=HP) + den_rep * bev_ref[...]
    attn = num / (den_rep + 1e-16)
    y = y_ref[...]
    x2 = attn + jnp.dot(y, Ws_ref[...], preferred_element_type=jnp.float32,
                        precision=HP) + bs_ref[...]
    xa = _ln(y + x2, g1_ref[...], be1_ref[...])
    hdn = jnp.maximum(
        jnp.dot(xa, W1_ref[...], preferred_element_type=jnp.float32,
                precision=HP) + b1_ref[...], 0.0)
    hdn = jnp.dot(hdn, W2_ref[...], preferred_element_type=jnp.float32,
                  precision=HP) + b2_ref[...]
    o_ref[...] = _ln(xa + hdn, g2_ref[...], be2_ref[...])


def _post(acc, den, y, Wz, S, bev, Ws_l, bs_l, W1_l, b1_l, W2_l, b2_l,
          g1_l, be1_l, g2_l, be2_l):
    row = pl.BlockSpec((ROWS, D), lambda i: (i, 0))
    den_spec = pl.BlockSpec((ROWS, H), lambda i: (i, 0))
    acc_spec = pl.BlockSpec((2, ROWS, AW), lambda i: (0, i, 0))
    full = lambda s: pl.BlockSpec(s, lambda i: (0,) * len(s))
    return pl.pallas_call(
        _post_body,
        grid=(N // ROWS,),
        in_specs=[acc_spec, den_spec, row, full((D, D)), full((H, D)),
                  full((D,)), full((D, D)), full((D,)), full((D, DFF)),
                  full((DFF,)), full((DFF, D)), full((D,)), full((D,)),
                  full((D,)), full((D,)), full((D,))],
        out_specs=row,
        out_shape=jax.ShapeDtypeStruct((N, D), jnp.float32),
    )(acc, den, y, Wz, S, bev, Ws_l, bs_l, W1_l, b1_l, W2_l, b2_l,
      g1_l, be1_l, g2_l, be2_l)


def _prep_layer(l, Wq, bq, Wk, bk, Wv, bv, We, be):
    Wq4 = (Wq[l] / 4.0).reshape(D, H, C)
    bq4 = (bq[l] / 4.0).reshape(H, C)
    Wer = We[l].reshape(DE, H, C)
    Wqe = jnp.einsum('ihc,dhc->ihd', Wq4, Wer, precision=HP)
    bqe = jnp.einsum('hc,dhc->hd', bq4, Wer, precision=HP)
    Wkr = Wk[l].reshape(D, H, C)
    Wvr = Wv[l].reshape(D, H, C)
    bkr = bk[l].reshape(H, C)
    bvr = bv[l].reshape(H, C)

    def cat(w4, b4):
        cols = []
        bs = []
        for c in range(NC):
            cols.append(w4[:, c * HC:(c + 1) * HC].reshape(D, HC * C))
            bs.append(b4[c * HC:(c + 1) * HC].reshape(HC * C))
        return cols, bs

    qc, qb = cat(Wq4, bq4)
    qec, qeb = cat(jnp.moveaxis(Wqe, 0, 0), bqe)
    kc, kb = cat(Wkr, bkr)
    vc, vb = cat(Wvr, bvr)
    Wcat = jnp.concatenate(
        [qc[0], qec[0], qc[1], qec[1], kc[0], vc[0], kc[1], vc[1]], axis=1)
    bcat = jnp.concatenate(
        [qb[0], qeb[0], qb[1], qeb[1], kb[0], vb[0], kb[1], vb[1]], axis=0)
    eye = jnp.eye(H, dtype=jnp.float32)
    Wz = jnp.einsum('dhc,hg->hdgc', Wer, eye).reshape(H * DE, H * C)
    return Wcat, bcat, Wz


def kernel(input, embedding, edge_attr, edge_index, W_in, b_in, W_emb, b_emb,
           Wq, bq, Wk, bk, Wv, bv, We, be, Ws, bs, W1, b1, W2, b2, g1, be1,
           g2, be2):
    S = jnp.repeat(jnp.eye(H, dtype=jnp.float32), C, axis=1)
    src = edge_index[0]
    dst = edge_index[1]
    inp, x = _proj(input, embedding, W_in, b_in, W_emb, b_emb)
    for l in range(L):
        Wcat, bcat, Wz = _prep_layer(l, Wq, bq, Wk, bk, Wv, bv, We, be)
        y, qq, kv = _tables(x, inp, Wcat, bcat)
        acc, den_raw = _edge_phase(qq.reshape(2 * N, D), kv.reshape(2 * N, D),
                                   src, dst, edge_attr)
        dp = den_raw[:, :N // 16].reshape(2, N // 16, 8, 2, 8)[..., :HC]
        den = dp.transpose(1, 2, 3, 0, 4).reshape(N, H)
        x = _post(acc, den, y, Wz, S, be[l], Ws[l], bs[l], W1[l], b1[l],
                  W2[l], b2[l], g1[l], be1[l], g2[l], be2[l])
    return x
